# baseline (speedup 1.0000x reference)
import jax
import jax.numpy as jnp
from jax.experimental import pallas as pl
from jax.experimental.pallas import tpu as pltpu

_BN = 512
_BB = 512


def _exp_slice(x, xx, g):
    gg = jnp.sum(g * g, axis=1)
    xg = jax.lax.dot_general(x, g, (((1,), (1,)), ((), ())),
                             preferred_element_type=jnp.float32)
    sq = jnp.maximum(xx - 2.0 * xg + gg[None, :], 0.0)
    return jnp.exp(-0.5 * sq).astype(jnp.bfloat16)


def _kern(x_ref, xn_ref, g_ref, c_ref, o_ref, kta_ref, ktb_ref, cb_ref):
    i = pl.program_id(0)
    j = pl.program_id(1)
    ni = pl.num_programs(0)
    nb = g_ref.shape[0] // _BB
    offs = [0]
    for t in range(nb):
        offs.append(offs[-1] + (nb - t) * _BB)
    par = jax.lax.rem(i, 2)

    @pl.when((i == 0) & (j == 0))
    def _init_kt0():
        x = x_ref[...]
        xx = jnp.sum(x * x, axis=1, keepdims=True)
        for k in range(nb):
            kta_ref[:, k * _BB:(k + 1) * _BB] = _exp_slice(
                x, xx, g_ref[k * _BB:(k + 1) * _BB, :])

    for jj in range(nb):
        @pl.when(j == jj)
        def _arm(jj=jj):
            lo = jj * _BB
            height = (nb - jj) * _BB
            off = offs[jj]

            @pl.when(i == 0)
            def _cast_panel():
                cb_ref[off:off + height, :] = (
                    c_ref[lo:, :].astype(jnp.bfloat16))

            def _dot(kt_ref):
                o_ref[...] = jnp.dot(
                    kt_ref[:, lo:],
                    cb_ref[off:off + height, :],
                    preferred_element_type=jnp.float32,
                )

            def _exp_next(ktn_ref):
                xn = xn_ref[...]
                xxn = jnp.sum(xn * xn, axis=1, keepdims=True)
                ktn_ref[:, lo:lo + _BB] = _exp_slice(
                    xn, xxn, g_ref[lo:lo + _BB, :])

            @pl.when(par == 0)
            def _even():
                _dot(kta_ref)

                @pl.when(i < ni - 1)
                def _():
                    _exp_next(ktb_ref)

            @pl.when(par == 1)
            def _odd():
                _dot(ktb_ref)

                @pl.when(i < ni - 1)
                def _():
                    _exp_next(kta_ref)


def kernel(input, sparse_grid, chol_inv):
    n, d = input.shape
    m = sparse_grid.shape[0]
    nb = m // _BB
    ni = n // _BN
    tri_rows = sum((nb - t) * _BB for t in range(nb))

    return pl.pallas_call(
        _kern,
        grid=(ni, nb),
        in_specs=[
            pl.BlockSpec((_BN, d), lambda i, j: (i, 0)),
            pl.BlockSpec((_BN, d),
                         lambda i, j: (jnp.minimum(i + 1, ni - 1), 0)),
            pl.BlockSpec((m, d), lambda i, j: (0, 0)),
            pl.BlockSpec((m, _BB),
                         lambda i, j: (0, jnp.where(i == 0, j, nb - 1))),
        ],
        out_specs=pl.BlockSpec((_BN, _BB), lambda i, j: (i, j)),
        out_shape=jax.ShapeDtypeStruct((n, m), jnp.float32),
        scratch_shapes=[
            pltpu.VMEM((_BN, m), jnp.bfloat16),
            pltpu.VMEM((_BN, m), jnp.bfloat16),
            pltpu.VMEM((tri_rows, _BB), jnp.bfloat16),
        ],
    )(input, input, sparse_grid, chol_inv)


# speedup vs baseline: 1.1774x; 1.1774x over previous
import jax
import jax.numpy as jnp
from jax.experimental import pallas as pl
from jax.experimental.pallas import tpu as pltpu

_BN = 1024
_BB = 512


def _kern(x_ref, g_ref, c_ref, o_ref, kt_ref, cb_ref):
    i = pl.program_id(0)
    j = pl.program_id(1)
    nb = g_ref.shape[0] // _BB
    offs = [0]
    for t in range(nb):
        offs.append(offs[-1] + (nb - t) * _BB)

    @pl.when(j == 0)
    def _compute_kt():
        x = x_ref[...]
        xx = jnp.sum(x * x, axis=1, keepdims=True)
        for k in range(nb):
            g = g_ref[k * _BB:(k + 1) * _BB, :]
            gg = jnp.sum(g * g, axis=1)
            xg = jax.lax.dot_general(x, g, (((1,), (1,)), ((), ())),
                                     preferred_element_type=jnp.float32)
            sq = jnp.maximum(xx - 2.0 * xg + gg[None, :], 0.0)
            kt_ref[:, k * _BB:(k + 1) * _BB] = (
                jnp.exp(-0.5 * sq).astype(jnp.bfloat16))

    for jj in range(nb):
        @pl.when(j == jj)
        def _panel(jj=jj):
            lo = jj * _BB
            height = (nb - jj) * _BB
            off = offs[jj]

            @pl.when(i == 0)
            def _cast_panel():
                cb_ref[off:off + height, :] = (
                    c_ref[lo:, :].astype(jnp.bfloat16))

            o_ref[...] = jnp.dot(
                kt_ref[:, lo:],
                cb_ref[off:off + height, :],
                preferred_element_type=jnp.float32,
            )


def kernel(input, sparse_grid, chol_inv):
    n, d = input.shape
    m = sparse_grid.shape[0]
    nb = m // _BB
    tri_rows = sum((nb - t) * _BB for t in range(nb))

    return pl.pallas_call(
        _kern,
        grid=(n // _BN, nb),
        in_specs=[
            pl.BlockSpec((_BN, d), lambda i, j: (i, 0)),
            pl.BlockSpec((m, d), lambda i, j: (0, 0)),
            pl.BlockSpec((m, _BB),
                         lambda i, j: (0, jnp.where(i == 0, j, nb - 1))),
        ],
        out_specs=pl.BlockSpec((_BN, _BB), lambda i, j: (i, j)),
        out_shape=jax.ShapeDtypeStruct((n, m), jnp.float32),
        scratch_shapes=[
            pltpu.VMEM((_BN, m), jnp.bfloat16),
            pltpu.VMEM((tri_rows, _BB), jnp.bfloat16),
        ],
    )(input, sparse_grid, chol_inv)
